# baseline (speedup 1.0000x reference)
import functools

import jax
import jax.numpy as jnp
from jax.experimental import pallas as pl


def _mlp_kernel(x_ref, W1_ref, b1_ref, W2_ref, b2_ref,
                Ws1_ref, bs1_ref, Ws2_ref, bs2_ref, x1_ref, xs_ref):
    x = x_ref[...]
    h1 = jax.nn.relu(jnp.dot(x, W1_ref[...], preferred_element_type=jnp.float32)
                     + b1_ref[...])
    x1_ref[...] = jax.nn.relu(
        jnp.dot(h1, W2_ref[...], preferred_element_type=jnp.float32) + b2_ref[...])
    hs = jax.nn.relu(jnp.dot(x, Ws1_ref[...], preferred_element_type=jnp.float32)
                     + bs1_ref[...])
    xs_ref[...] = jax.nn.relu(
        jnp.dot(hs, Ws2_ref[...], preferred_element_type=jnp.float32) + bs2_ref[...])


def _msg_kernel(A_ref, W_ref, x1_ref, xs_ref, E_ref, out_ref, *, f, ch):
    a = A_ref[0]
    nf = W_ref.shape[2]
    asum = jnp.sum(a, axis=1, keepdims=True)
    scale = jnp.where(asum != 0.0, 1.0 / asum, 0.0)
    emat = E_ref[...]
    acc = jnp.zeros((a.shape[0], 128), jnp.float32)
    for c in range(nf // ch):
        w = W_ref[0, :, c * ch:(c + 1) * ch]
        xv = x1_ref[0, :, c * ch:(c + 1) * ch]
        ac = a[:, c * (ch // f):(c + 1) * (ch // f)]
        ar = jnp.dot(ac, emat, preferred_element_type=jnp.float32)
        t = w * xv * ar
        width = ch
        while width > 128:
            half = width // 2
            t = t[:, :half] + t[:, half:width]
            width = half
        acc = acc + t
    r = acc[:, 0:f]
    for g in range(1, 128 // f):
        r = r + acc[:, g * f:(g + 1) * f]
    out_ref[0] = r * scale + xs_ref[0]


@jax.jit
def kernel(A, W, x, W1, b1, W2, b2, Ws1, bs1, Ws2, bs2):
    b, n, din = x.shape
    f = W.shape[-1]

    x2d = x.reshape(b * n, din)
    x1f, xsf = pl.pallas_call(
        _mlp_kernel,
        out_shape=(
            jax.ShapeDtypeStruct((b * n, f), jnp.float32),
            jax.ShapeDtypeStruct((b * n, f), jnp.float32),
        ),
    )(x2d, W1, b1.reshape(1, f), W2, b2.reshape(1, f),
      Ws1, bs1.reshape(1, f), Ws2, bs2.reshape(1, f))
    x1 = x1f.reshape(b, 1, n * f)
    xs = xsf.reshape(b, n, f)

    ch = 512
    E = (jnp.arange(ch, dtype=jnp.int32)[None, :] // f
         == jnp.arange(ch // f, dtype=jnp.int32)[:, None]).astype(jnp.float32)

    Wf = W.reshape(b, n, n * f)
    BI = 128
    x2 = pl.pallas_call(
        functools.partial(_msg_kernel, f=f, ch=ch),
        grid=(b, n // BI),
        in_specs=[
            pl.BlockSpec((1, BI, n), lambda bb, ii: (bb, ii, 0)),
            pl.BlockSpec((1, BI, n * f), lambda bb, ii: (bb, ii, 0)),
            pl.BlockSpec((1, 1, n * f), lambda bb, ii: (bb, 0, 0)),
            pl.BlockSpec((1, BI, f), lambda bb, ii: (bb, ii, 0)),
            pl.BlockSpec((ch // f, ch), lambda bb, ii: (0, 0)),
        ],
        out_specs=pl.BlockSpec((1, BI, f), lambda bb, ii: (bb, ii, 0)),
        out_shape=jax.ShapeDtypeStruct((b, n, f), jnp.float32),
    )(A, Wf, x1, xs, E)

    return (W, x2)
